# Initial kernel scaffold; baseline (speedup 1.0000x reference)
import functools

import jax
import jax.numpy as jnp
from jax import lax
from jax.experimental import pallas as pl
from jax.experimental.pallas import tpu as pltpu
from jax.experimental.pallas import tpu_sc as plsc

P = 262144
K = 1000000
D = 32
NC = 2
NS = 16
NW = NC * NS
PW = P // NW
C = 16
G = C * 8
NCHUNK = PW // C
NLOOP = NCHUNK // 2

_mesh = plsc.VectorSubcoreMesh(core_axis_name="c", subcore_axis_name="s")


@functools.partial(
    pl.kernel,
    mesh=_mesh,
    out_type=jax.ShapeDtypeStruct((P, D), jnp.float32),
    scratch_types=[
        pltpu.VMEM((2, G), jnp.int32),
        pltpu.VMEM((2, 3 * C), jnp.float32),
        pltpu.VMEM((2, G, D), jnp.float32),
        pltpu.VMEM((2, C, D), jnp.float32),
        pltpu.SemaphoreType.DMA,
        pltpu.SemaphoreType.DMA,
        pltpu.SemaphoreType.DMA,
        pltpu.SemaphoreType.DMA,
        pltpu.SemaphoreType.DMA,
        pltpu.SemaphoreType.DMA,
        pltpu.SemaphoreType.DMA,
        pltpu.SemaphoreType.DMA,
    ],
)
def _voxel_interp(feats_hbm, p_hbm, table_hbm, out_hbm,
                  idx_v, p_v, rows_v, out_v,
                  isem0, isem1, psem0, psem1, gsem0, gsem1, osem0, osem1):
    isem = (isem0, isem1)
    psem = (psem0, psem1)
    gsem = (gsem0, gsem1)
    osem = (osem0, osem1)
    wid = lax.axis_index("s") * NC + lax.axis_index("c")
    base0 = wid * PW

    def idx_copy(g, b):
        return pltpu.make_async_copy(
            feats_hbm.at[pl.ds((base0 + g * C) * 8, G)], idx_v.at[b], isem[b])

    def p_copy(g, b):
        return pltpu.make_async_copy(
            p_hbm.at[pl.ds((base0 + g * C) * 3, 3 * C)], p_v.at[b], psem[b])

    def gather_copy(b):
        return pltpu.make_async_copy(
            table_hbm.at[idx_v.at[b]], rows_v.at[b], gsem[b])

    def out_copy(g, b):
        return pltpu.make_async_copy(
            out_v.at[b], out_hbm.at[pl.ds(base0 + g * C, C)], osem[b])

    def compute(b):
        for i in range(C):
            px = p_v[b, 3 * i]
            py = p_v[b, 3 * i + 1]
            pz = p_v[b, 3 * i + 2]
            wx = (1.0 - px, px)
            wy = (1.0 - py, py)
            wz = (1.0 - pz, pz)
            wxy = (wx[0] * wy[0], wx[0] * wy[1], wx[1] * wy[0], wx[1] * wy[1])
            acc0 = None
            acc1 = None
            for c in range(8):
                w = wxy[c >> 1] * wz[c & 1]
                t0 = w * rows_v[b, 8 * i + c, pl.ds(0, 16)]
                t1 = w * rows_v[b, 8 * i + c, pl.ds(16, 16)]
                acc0 = t0 if acc0 is None else acc0 + t0
                acc1 = t1 if acc1 is None else acc1 + t1
            out_v[b, i, pl.ds(0, 16)] = acc0
            out_v[b, i, pl.ds(16, 16)] = acc1

    idx_copy(0, 0).start()
    idx_copy(1, 1).start()
    p_copy(0, 0).start()
    p_copy(1, 1).start()
    idx_copy(0, 0).wait()
    gather_copy(0).start()
    idx_copy(1, 1).wait()
    gather_copy(1).start()

    def loop_body(it, carry):
        for b in range(2):
            g = 2 * it + b
            gather_copy(b).wait()

            @pl.when(g + 2 < NCHUNK)
            def _():
                idx_copy(g + 2, b).start()

            p_copy(g, b).wait()

            @pl.when(it > 0)
            def _():
                out_copy(g, b).wait()

            compute(b)
            out_copy(g, b).start()

            @pl.when(g + 2 < NCHUNK)
            def _():
                p_copy(g + 2, b).start()
                idx_copy(g + 2, b).wait()
                gather_copy(b).start()

        return carry

    lax.fori_loop(0, NLOOP, loop_body, 0)
    out_copy(0, 0).wait()
    out_copy(0, 1).wait()


def kernel(feats, p, values_weight):
    feats_flat = feats.reshape(-1).astype(jnp.int32)
    p_flat = p.reshape(-1).astype(jnp.float32)
    return _voxel_interp(feats_flat, p_flat, values_weight)


# speedup vs baseline: 3.0923x; 3.0923x over previous
import functools

import jax
import jax.numpy as jnp
from jax import lax
from jax.experimental import pallas as pl
from jax.experimental.pallas import tpu as pltpu
from jax.experimental.pallas import tpu_sc as plsc

P = 262144
K = 1000000
D = 32
NC = 2
NS = 16
NW = NC * NS
PW = P // NW
C = 16
G = C * 8
NCHUNK = PW // C
NLOOP = NCHUNK // 2

_mesh = plsc.VectorSubcoreMesh(core_axis_name="c", subcore_axis_name="s")


@functools.partial(
    pl.kernel,
    mesh=_mesh,
    out_type=jax.ShapeDtypeStruct((P, D), jnp.float32),
    compiler_params=pltpu.CompilerParams(use_tc_tiling_on_sc=False),
    scratch_types=[
        pltpu.VMEM((2, G), jnp.int32),
        pltpu.VMEM((2, 3 * C), jnp.float32),
        pltpu.VMEM((2, G, D), jnp.float32),
        pltpu.VMEM((2, C, D), jnp.float32),
        pltpu.SemaphoreType.DMA,
        pltpu.SemaphoreType.DMA,
        pltpu.SemaphoreType.DMA,
        pltpu.SemaphoreType.DMA,
        pltpu.SemaphoreType.DMA,
        pltpu.SemaphoreType.DMA,
        pltpu.SemaphoreType.DMA,
        pltpu.SemaphoreType.DMA,
    ],
)
def _voxel_interp(feats_hbm, p_hbm, table_hbm, out_hbm,
                  idx_v, p_v, rows_v, out_v,
                  isem0, isem1, psem0, psem1, gsem0, gsem1, osem0, osem1):
    isem = (isem0, isem1)
    psem = (psem0, psem1)
    gsem = (gsem0, gsem1)
    osem = (osem0, osem1)
    wid = lax.axis_index("s") * NC + lax.axis_index("c")
    base0 = wid * PW

    def idx_copy(g, b):
        return pltpu.make_async_copy(
            feats_hbm.at[pl.ds((base0 + g * C) * 8, G)], idx_v.at[b], isem[b])

    def p_copy(g, b):
        return pltpu.make_async_copy(
            p_hbm.at[pl.ds((base0 + g * C) * 3, 3 * C)], p_v.at[b], psem[b])

    def gather_copy(b):
        return pltpu.make_async_copy(
            table_hbm.at[idx_v.at[b]], rows_v.at[b], gsem[b])

    def out_copy(g, b):
        return pltpu.make_async_copy(
            out_v.at[b], out_hbm.at[pl.ds(base0 + g * C, C)], osem[b])

    def compute(b):
        pvec = (p_v[b, pl.ds(0, 16)], p_v[b, pl.ds(16, 16)],
                p_v[b, pl.ds(32, 16)])
        for i in range(C):
            px = pvec[(3 * i) // 16][(3 * i) % 16]
            py = pvec[(3 * i + 1) // 16][(3 * i + 1) % 16]
            pz = pvec[(3 * i + 2) // 16][(3 * i + 2) % 16]
            wx = (1.0 - px, px)
            wy = (1.0 - py, py)
            wz = (1.0 - pz, pz)
            wxy = (wx[0] * wy[0], wx[0] * wy[1], wx[1] * wy[0], wx[1] * wy[1])
            acc0 = None
            acc1 = None
            for c in range(8):
                w = wxy[c >> 1] * wz[c & 1]
                t0 = w * rows_v[b, 8 * i + c, pl.ds(0, 16)]
                t1 = w * rows_v[b, 8 * i + c, pl.ds(16, 16)]
                acc0 = t0 if acc0 is None else acc0 + t0
                acc1 = t1 if acc1 is None else acc1 + t1
            out_v[b, i, pl.ds(0, 16)] = acc0
            out_v[b, i, pl.ds(16, 16)] = acc1

    idx_copy(0, 0).start()
    idx_copy(1, 1).start()
    p_copy(0, 0).start()
    p_copy(1, 1).start()
    idx_copy(0, 0).wait()
    gather_copy(0).start()
    idx_copy(1, 1).wait()
    gather_copy(1).start()

    def loop_body(it, carry):
        for b in range(2):
            g = 2 * it + b
            gather_copy(b).wait()

            @pl.when(g + 2 < NCHUNK)
            def _():
                idx_copy(g + 2, b).start()

            p_copy(g, b).wait()

            @pl.when(it > 0)
            def _():
                out_copy(g, b).wait()

            compute(b)
            out_copy(g, b).start()

            @pl.when(g + 2 < NCHUNK)
            def _():
                p_copy(g + 2, b).start()
                idx_copy(g + 2, b).wait()
                gather_copy(b).start()

        return carry

    lax.fori_loop(0, NLOOP, loop_body, 0)
    out_copy(0, 0).wait()
    out_copy(0, 1).wait()


def kernel(feats, p, values_weight):
    feats_flat = feats.reshape(-1).astype(jnp.int32)
    p_flat = p.reshape(-1).astype(jnp.float32)
    return _voxel_interp(feats_flat, p_flat, values_weight)


# speedup vs baseline: 3.3821x; 1.0937x over previous
import functools

import jax
import jax.numpy as jnp
from jax import lax
from jax.experimental import pallas as pl
from jax.experimental.pallas import tpu as pltpu
from jax.experimental.pallas import tpu_sc as plsc

P = 262144
K = 1000000
D = 32
NC = 2
NS = 16
NW = NC * NS
PW = P // NW
C = 16
G = C * 8
NCHUNK = PW // C
NLOOP = NCHUNK // 2

_mesh = plsc.VectorSubcoreMesh(core_axis_name="c", subcore_axis_name="s")


@functools.partial(
    pl.kernel,
    mesh=_mesh,
    out_type=jax.ShapeDtypeStruct((P, D), jnp.float32),
    compiler_params=pltpu.CompilerParams(
        use_tc_tiling_on_sc=False, needs_layout_passes=False),
    scratch_types=[
        pltpu.VMEM((2, C, 8), jnp.int32),
        pltpu.VMEM((2, G), jnp.int32),
        pltpu.VMEM((2, 3, C), jnp.float32),
        pltpu.VMEM((2, G, D), jnp.float32),
        pltpu.VMEM((2, C, D), jnp.float32),
        pltpu.SemaphoreType.DMA,
        pltpu.SemaphoreType.DMA,
        pltpu.SemaphoreType.DMA,
        pltpu.SemaphoreType.DMA,
        pltpu.SemaphoreType.DMA,
        pltpu.SemaphoreType.DMA,
        pltpu.SemaphoreType.DMA,
        pltpu.SemaphoreType.DMA,
    ],
)
def _voxel_interp(feats_hbm, pt_hbm, table_hbm, out_hbm,
                  fst_v, idx_v, p_v, rows_v, out_v,
                  fsem0, fsem1, psem0, psem1, gsem0, gsem1, osem0, osem1):
    fsem = (fsem0, fsem1)
    psem = (psem0, psem1)
    gsem = (gsem0, gsem1)
    osem = (osem0, osem1)
    wid = lax.axis_index("s") * NC + lax.axis_index("c")
    base0 = wid * PW
    iota = lax.iota(jnp.int32, 16)

    def feats_copy(g, b):
        return pltpu.make_async_copy(
            feats_hbm.at[pl.ds(base0 + g * C, C), :], fst_v.at[b], fsem[b])

    def p_copy(g, b, j):
        return pltpu.make_async_copy(
            pt_hbm.at[j, pl.ds(base0 + g * C, C)], p_v.at[b, j], psem[b])

    def gather_copy(b):
        return pltpu.make_async_copy(
            table_hbm.at[idx_v.at[b]], rows_v.at[b], gsem[b])

    def out_copy(g, b):
        return pltpu.make_async_copy(
            out_v.at[b], out_hbm.at[pl.ds(base0 + g * C, C)], osem[b])

    def build_idx(b):
        for c in range(8):
            col = plsc.load_gather(
                fst_v.at[b], [iota, jnp.full((16,), c, jnp.int32)])
            idx_v[b, pl.ds(c * 16, 16)] = col

    def compute(b):
        px = p_v[b, 0, pl.ds(0, 16)]
        py = p_v[b, 1, pl.ds(0, 16)]
        pz = p_v[b, 2, pl.ds(0, 16)]
        wx = (1.0 - px, px)
        wy = (1.0 - py, py)
        wz = (1.0 - pz, pz)
        wxy = (wx[0] * wy[0], wx[0] * wy[1], wx[1] * wy[0], wx[1] * wy[1])
        wvec = tuple(wxy[c >> 1] * wz[c & 1] for c in range(8))
        for i in range(C):
            acc0 = None
            acc1 = None
            for c in range(8):
                w = wvec[c][i]
                t0 = w * rows_v[b, c * 16 + i, pl.ds(0, 16)]
                t1 = w * rows_v[b, c * 16 + i, pl.ds(16, 16)]
                acc0 = t0 if acc0 is None else acc0 + t0
                acc1 = t1 if acc1 is None else acc1 + t1
            out_v[b, i, pl.ds(0, 16)] = acc0
            out_v[b, i, pl.ds(16, 16)] = acc1

    for b in range(2):
        feats_copy(b, b).start()
        for j in range(3):
            p_copy(b, b, j).start()
    for b in range(2):
        feats_copy(b, b).wait()
        build_idx(b)
        gather_copy(b).start()

    def loop_body(it, carry):
        for b in range(2):
            g = 2 * it + b
            gather_copy(b).wait()

            @pl.when(g + 2 < NCHUNK)
            def _():
                feats_copy(g + 2, b).start()

            for j in range(3):
                p_copy(g, b, j).wait()

            @pl.when(it > 0)
            def _():
                out_copy(g, b).wait()

            compute(b)
            out_copy(g, b).start()

            @pl.when(g + 2 < NCHUNK)
            def _():
                for j in range(3):
                    p_copy(g + 2, b, j).start()
                feats_copy(g + 2, b).wait()
                build_idx(b)
                gather_copy(b).start()

        return carry

    lax.fori_loop(0, NLOOP, loop_body, 0)
    out_copy(0, 0).wait()
    out_copy(0, 1).wait()


def kernel(feats, p, values_weight):
    return _voxel_interp(feats, p.T, values_weight)


# speedup vs baseline: 3.7545x; 1.1101x over previous
import functools

import jax
import jax.numpy as jnp
from jax import lax
from jax.experimental import pallas as pl
from jax.experimental.pallas import tpu as pltpu
from jax.experimental.pallas import tpu_sc as plsc

P = 262144
K = 1000000
D = 32
NC = 2
NS = 16
NW = NC * NS
PW = P // NW
C = 16
G = C * 8
NCHUNK = PW // C
NBUF = 4
NLOOP = NCHUNK // NBUF

_mesh = plsc.VectorSubcoreMesh(core_axis_name="c", subcore_axis_name="s")


@functools.partial(
    pl.kernel,
    mesh=_mesh,
    out_type=jax.ShapeDtypeStruct((P, D), jnp.float32),
    compiler_params=pltpu.CompilerParams(
        use_tc_tiling_on_sc=False, needs_layout_passes=False),
    scratch_types=[
        pltpu.VMEM((NBUF, G), jnp.int32),
        pltpu.VMEM((NBUF, 3, C), jnp.float32),
        pltpu.VMEM((NBUF, G, D), jnp.float32),
        pltpu.VMEM((NBUF, C, D), jnp.float32),
        pltpu.SemaphoreType.DMA,
        pltpu.SemaphoreType.DMA,
        pltpu.SemaphoreType.DMA,
        pltpu.SemaphoreType.DMA,
        pltpu.SemaphoreType.DMA,
        pltpu.SemaphoreType.DMA,
        pltpu.SemaphoreType.DMA,
        pltpu.SemaphoreType.DMA,
        pltpu.SemaphoreType.DMA,
        pltpu.SemaphoreType.DMA,
        pltpu.SemaphoreType.DMA,
        pltpu.SemaphoreType.DMA,
        pltpu.SemaphoreType.DMA,
        pltpu.SemaphoreType.DMA,
        pltpu.SemaphoreType.DMA,
        pltpu.SemaphoreType.DMA,
    ],
)
def _voxel_interp(featsT_hbm, pt_hbm, table_hbm, out_hbm,
                  idx_v, p_v, rows_v, out_v,
                  i0, i1, i2, i3, p0, p1, p2, p3,
                  g0, g1, g2, g3, o0, o1, o2, o3):
    isem = (i0, i1, i2, i3)
    psem = (p0, p1, p2, p3)
    gsem = (g0, g1, g2, g3)
    osem = (o0, o1, o2, o3)
    wid = lax.axis_index("s") * NC + lax.axis_index("c")
    base0 = wid * PW

    def idx_copy(g, b, c):
        return pltpu.make_async_copy(
            featsT_hbm.at[c, pl.ds(base0 + g * C, C)],
            idx_v.at[b, pl.ds(c * C, C)], isem[b])

    def p_copy(g, b, j):
        return pltpu.make_async_copy(
            pt_hbm.at[j, pl.ds(base0 + g * C, C)], p_v.at[b, j], psem[b])

    def gather_copy(b):
        return pltpu.make_async_copy(
            table_hbm.at[idx_v.at[b]], rows_v.at[b], gsem[b])

    def out_copy(g, b):
        return pltpu.make_async_copy(
            out_v.at[b], out_hbm.at[pl.ds(base0 + g * C, C)], osem[b])

    def stage_in(g, b):
        for c in range(8):
            idx_copy(g, b, c).start()
        for j in range(3):
            p_copy(g, b, j).start()

    def wait_idx(b):
        for c in range(8):
            idx_copy(0, b, c).wait()

    def compute(b):
        px = p_v[b, 0, pl.ds(0, 16)]
        py = p_v[b, 1, pl.ds(0, 16)]
        pz = p_v[b, 2, pl.ds(0, 16)]
        wx = (1.0 - px, px)
        wy = (1.0 - py, py)
        wz = (1.0 - pz, pz)
        wxy = (wx[0] * wy[0], wx[0] * wy[1], wx[1] * wy[0], wx[1] * wy[1])
        wvec = tuple(wxy[c >> 1] * wz[c & 1] for c in range(8))
        for i in range(C):
            acc0 = None
            acc1 = None
            for c in range(8):
                w = wvec[c][i]
                t0 = w * rows_v[b, c * C + i, pl.ds(0, 16)]
                t1 = w * rows_v[b, c * C + i, pl.ds(16, 16)]
                acc0 = t0 if acc0 is None else acc0 + t0
                acc1 = t1 if acc1 is None else acc1 + t1
            out_v[b, i, pl.ds(0, 16)] = acc0
            out_v[b, i, pl.ds(16, 16)] = acc1

    for b in range(NBUF):
        stage_in(b, b)
    for b in range(NBUF):
        wait_idx(b)
        gather_copy(b).start()

    def loop_body(it, carry):
        for b in range(NBUF):
            g = NBUF * it + b
            gather_copy(b).wait()

            @pl.when(g + NBUF < NCHUNK)
            def _():
                for c in range(8):
                    idx_copy(g + NBUF, b, c).start()

            for j in range(3):
                p_copy(g, b, j).wait()

            @pl.when(it > 0)
            def _():
                out_copy(g, b).wait()

            compute(b)
            out_copy(g, b).start()

            @pl.when(g + NBUF < NCHUNK)
            def _():
                for j in range(3):
                    p_copy(g + NBUF, b, j).start()
                wait_idx(b)
                gather_copy(b).start()

        return carry

    lax.fori_loop(0, NLOOP, loop_body, 0)
    for b in range(NBUF):
        out_copy(0, b).wait()


def kernel(feats, p, values_weight):
    return _voxel_interp(feats.T, p.T, values_weight)
